# Initial kernel scaffold; baseline (speedup 1.0000x reference)
import jax
import jax.numpy as jnp
from jax.experimental import pallas as pl


def kernel(x, gate_w, ew1, ew2, ew3, sw1, sw2, sw3):
    raise NotImplementedError("write your pallas kernel here")


# speedup vs baseline: 1.2958x; 1.2958x over previous
import functools

import jax
import jax.numpy as jnp
from jax import lax
from jax.experimental import pallas as pl

DIM = 1024
INTER = 512
N_EXPERTS = 8
TOP_K = 2
N_SHARED = 2
T = 2048
TB = 256
S_INTER = INTER * N_SHARED


def _gate_shared_body(x_ref, gw_ref, sw1_ref, sw2_ref, sw3_ref, w_ref, ys_ref):
    x = x_ref[...]
    logits = lax.dot_general(x, gw_ref[...], (((1,), (1,)), ((), ())),
                             preferred_element_type=jnp.float32)
    s = jax.nn.sigmoid(logits)
    iota = lax.broadcasted_iota(jnp.int32, s.shape, 1)
    m1 = jnp.max(s, axis=1, keepdims=True)
    i1 = jnp.min(jnp.where(s == m1, iota, N_EXPERTS), axis=1, keepdims=True)
    s2 = jnp.where(iota == i1, -jnp.inf, s)
    m2 = jnp.max(s2, axis=1, keepdims=True)
    i2 = jnp.min(jnp.where(s2 == m2, iota, N_EXPERTS), axis=1, keepdims=True)
    denom = m1 + m2
    w_ref[...] = (jnp.where(iota == i1, m1 / denom, 0.0)
                  + jnp.where(iota == i2, m2 / denom, 0.0))
    h1 = lax.dot_general(x, sw1_ref[...], (((1,), (1,)), ((), ())),
                         preferred_element_type=jnp.float32)
    h3 = lax.dot_general(x, sw3_ref[...], (((1,), (1,)), ((), ())),
                         preferred_element_type=jnp.float32)
    h = h1 * jax.nn.sigmoid(h1) * h3
    ys_ref[...] = lax.dot_general(h, sw2_ref[...], (((1,), (1,)), ((), ())),
                                  preferred_element_type=jnp.float32)


def _experts_body(x_ref, w_ref, ys_ref, ew1_ref, ew2_ref, ew3_ref, o_ref):
    e = pl.program_id(1)
    x = x_ref[...]
    h1 = lax.dot_general(x, ew1_ref[0], (((1,), (1,)), ((), ())),
                         preferred_element_type=jnp.float32)
    h3 = lax.dot_general(x, ew3_ref[0], (((1,), (1,)), ((), ())),
                         preferred_element_type=jnp.float32)
    h = h1 * jax.nn.sigmoid(h1) * h3
    y = lax.dot_general(h, ew2_ref[0], (((1,), (1,)), ((), ())),
                        preferred_element_type=jnp.float32)
    iota = lax.broadcasted_iota(jnp.int32, (TB, N_EXPERTS), 1)
    wi = jnp.sum(jnp.where(iota == e, w_ref[...], 0.0), axis=1, keepdims=True)
    yw = y * wi

    @pl.when(e == 0)
    def _():
        o_ref[...] = ys_ref[...] + yw

    @pl.when(e > 0)
    def _():
        o_ref[...] = o_ref[...] + yw


def kernel(x, gate_w, ew1, ew2, ew3, sw1, sw2, sw3):
    shape = x.shape
    xf = x.reshape(-1, DIM)
    nb = T // TB

    wfull, ys = pl.pallas_call(
        _gate_shared_body,
        grid=(nb,),
        in_specs=[
            pl.BlockSpec((TB, DIM), lambda i: (i, 0)),
            pl.BlockSpec((N_EXPERTS, DIM), lambda i: (0, 0)),
            pl.BlockSpec((S_INTER, DIM), lambda i: (0, 0)),
            pl.BlockSpec((DIM, S_INTER), lambda i: (0, 0)),
            pl.BlockSpec((S_INTER, DIM), lambda i: (0, 0)),
        ],
        out_specs=[
            pl.BlockSpec((TB, N_EXPERTS), lambda i: (i, 0)),
            pl.BlockSpec((TB, DIM), lambda i: (i, 0)),
        ],
        out_shape=[
            jax.ShapeDtypeStruct((T, N_EXPERTS), jnp.float32),
            jax.ShapeDtypeStruct((T, DIM), jnp.float32),
        ],
    )(xf, gate_w, sw1, sw2, sw3)

    y = pl.pallas_call(
        _experts_body,
        grid=(nb, N_EXPERTS),
        in_specs=[
            pl.BlockSpec((TB, DIM), lambda i, e: (i, 0)),
            pl.BlockSpec((TB, N_EXPERTS), lambda i, e: (i, 0)),
            pl.BlockSpec((TB, DIM), lambda i, e: (i, 0)),
            pl.BlockSpec((1, INTER, DIM), lambda i, e: (e, 0, 0)),
            pl.BlockSpec((1, DIM, INTER), lambda i, e: (e, 0, 0)),
            pl.BlockSpec((1, INTER, DIM), lambda i, e: (e, 0, 0)),
        ],
        out_specs=pl.BlockSpec((TB, DIM), lambda i, e: (i, 0)),
        out_shape=jax.ShapeDtypeStruct((T, DIM), jnp.float32),
    )(xf, wfull, ys, ew1, ew2, ew3)

    return y.reshape(shape)
